# Initial kernel scaffold; baseline (speedup 1.0000x reference)
import jax
import jax.numpy as jnp
from jax.experimental import pallas as pl


def kernel(features, edge_index, adj_values, W, bias, prelu_a):
    raise NotImplementedError("write your pallas kernel here")


# speedup vs baseline: 1.7074x; 1.7074x over previous
import functools

import jax
import jax.numpy as jnp
from jax import lax
from jax.experimental import pallas as pl
from jax.experimental.pallas import tpu as pltpu
from jax.experimental.pallas import tpu_sc as plsc

N = 10000
E = 160000
D = 256
DH = 128
NS = 16
EPS = E // NS
CH = 80
NCHUNK = EPS // CH
ROWS_PS = N // NS
ZROWS = 25
MM_BLK = 1000


def _mm_body(x_ref, w_ref, o_ref):
    r = lax.dot_general(x_ref[...], w_ref[...], (((1,), (1,)), ((), ())),
                        preferred_element_type=jnp.float32)
    o_ref[0] = r[:, :DH]
    o_ref[1] = r[:, DH:]


def _matmul_halves(features, W):
    return pl.pallas_call(
        _mm_body,
        grid=(N // MM_BLK,),
        in_specs=[
            pl.BlockSpec((MM_BLK, D), lambda i: (i, 0)),
            pl.BlockSpec((D, D), lambda i: (0, 0)),
        ],
        out_specs=pl.BlockSpec((2, MM_BLK, DH), lambda i: (0, i, 0)),
        out_shape=jax.ShapeDtypeStruct((2, N, DH), jnp.float32),
    )(features, W)


def _sc_body(seq_hbm, src_hbm, dst_hbm, adj_hbm, out_hbm,
             src_v, dst_v, adj_v, rows_v, msgs_v, zbuf_v, agg_s, sem):
    c = lax.axis_index("c")
    s = lax.axis_index("s")

    def zrow(r, carry):
        for j in range(DH // 16):
            zbuf_v[r, pl.ds(j * 16, 16)] = jnp.zeros((16,), jnp.float32)
        return carry
    lax.fori_loop(0, ZROWS, zrow, 0)
    for k in range(ROWS_PS // ZROWS):
        pltpu.sync_copy(zbuf_v, agg_s.at[pl.ds(s * ROWS_PS + k * ZROWS, ZROWS)])
    plsc.subcore_barrier()

    def core_body(seq_plane):
        def chunk(i, carry):
            row = s * NCHUNK + i
            pltpu.sync_copy(src_hbm.at[row], src_v)
            pltpu.sync_copy(dst_hbm.at[row], dst_v)
            pltpu.sync_copy(adj_hbm.at[row], adj_v)
            pltpu.async_copy(seq_plane.at[src_v], rows_v, sem).wait()

            def edge(e, carry2):
                a = plsc.load_gather(adj_v, [jnp.full((16,), e, jnp.int32)])
                for j in range(DH // 16):
                    sl = pl.ds(j * 16, 16)
                    msgs_v[e, sl] = rows_v[e, sl] * a
                return carry2
            lax.fori_loop(0, CH, edge, 0)
            pltpu.sync_copy(msgs_v, agg_s.at[dst_v], add=True)
            return carry
        lax.fori_loop(0, NCHUNK, chunk, 0)

    @pl.when(c == 0)
    def _():
        core_body(seq_hbm.at[0])

    @pl.when(c == 1)
    def _():
        core_body(seq_hbm.at[1])

    plsc.subcore_barrier()

    @pl.when(c == 0)
    def _():
        pltpu.sync_copy(agg_s.at[pl.ds(s * ROWS_PS, ROWS_PS)],
                        out_hbm.at[0].at[pl.ds(s * ROWS_PS, ROWS_PS)])

    @pl.when(c == 1)
    def _():
        pltpu.sync_copy(agg_s.at[pl.ds(s * ROWS_PS, ROWS_PS)],
                        out_hbm.at[1].at[pl.ds(s * ROWS_PS, ROWS_PS)])


def _sc_aggregate(seq, src, dst, adj):
    mesh = plsc.VectorSubcoreMesh(core_axis_name="c", subcore_axis_name="s")
    return pl.kernel(
        _sc_body,
        out_type=jax.ShapeDtypeStruct((2, N, DH), jnp.float32),
        mesh=mesh,
        scratch_types=[
            pltpu.VMEM((CH,), jnp.int32),
            pltpu.VMEM((CH,), jnp.int32),
            pltpu.VMEM((CH,), jnp.float32),
            pltpu.VMEM((CH, DH), jnp.float32),
            pltpu.VMEM((CH, DH), jnp.float32),
            pltpu.VMEM((ZROWS, DH), jnp.float32),
            pltpu.VMEM_SHARED((N, DH), jnp.float32),
            pltpu.SemaphoreType.DMA,
        ],
        compiler_params=pltpu.CompilerParams(use_tc_tiling_on_sc=False,
                                             needs_layout_passes=False),
    )(seq, src, dst, adj)


def _act_body(agg_ref, b_ref, a_ref, o_ref):
    x = jnp.concatenate([agg_ref[0], agg_ref[1]], axis=1) + b_ref[...]
    alpha = a_ref[0]
    o_ref[...] = jnp.where(x >= 0, x, alpha * x)


def _epilogue(agg, bias, prelu_a):
    return pl.pallas_call(
        _act_body,
        grid=(N // MM_BLK,),
        in_specs=[
            pl.BlockSpec((2, MM_BLK, DH), lambda i: (0, i, 0)),
            pl.BlockSpec((1, D), lambda i: (0, 0)),
            pl.BlockSpec(memory_space=pltpu.SMEM),
        ],
        out_specs=pl.BlockSpec((MM_BLK, D), lambda i: (i, 0)),
        out_shape=jax.ShapeDtypeStruct((N, D), jnp.float32),
    )(agg, bias.reshape(1, D), prelu_a)


@jax.jit
def kernel(features, edge_index, adj_values, W, bias, prelu_a):
    src = edge_index[0].reshape(NS * NCHUNK, CH)
    dst = edge_index[1].reshape(NS * NCHUNK, CH)
    adj = adj_values.reshape(NS * NCHUNK, CH)
    seq = _matmul_halves(features, W)
    agg = _sc_aggregate(seq, src, dst, adj)
    return _epilogue(agg, bias, prelu_a)


# speedup vs baseline: 5.9932x; 3.5101x over previous
import functools

import jax
import jax.numpy as jnp
from jax import lax
from jax.experimental import pallas as pl
from jax.experimental.pallas import tpu as pltpu
from jax.experimental.pallas import tpu_sc as plsc

N = 10000
E = 160000
D = 256
DH = 128
NS = 16
EPS = E // NS
CH = 80
NCHUNK = EPS // CH
ROWS_PS = N // NS
ZROWS = 25
MM_BLK = 1000


def _mm_body(x_ref, w_ref, o_ref):
    r = lax.dot_general(x_ref[...], w_ref[...], (((1,), (1,)), ((), ())),
                        preferred_element_type=jnp.float32)
    o_ref[0] = r[:, :DH]
    o_ref[1] = r[:, DH:]


def _matmul_halves(features, W):
    return pl.pallas_call(
        _mm_body,
        grid=(N // MM_BLK,),
        in_specs=[
            pl.BlockSpec((MM_BLK, D), lambda i: (i, 0)),
            pl.BlockSpec((D, D), lambda i: (0, 0)),
        ],
        out_specs=pl.BlockSpec((2, MM_BLK, DH), lambda i: (0, i, 0)),
        out_shape=jax.ShapeDtypeStruct((2, N, DH), jnp.float32),
    )(features, W)


def _sc_body(seq_hbm, src_hbm, dst_hbm, adj_hbm, out_hbm,
             src_v, dst_v, adj_v, rows0_v, rows1_v, agg_s, sem):
    c = lax.axis_index("c")
    s = lax.axis_index("s")

    def zrow(r, carry):
        for j in range(DH // 16):
            rows0_v[r, pl.ds(j * 16, 16)] = jnp.zeros((16,), jnp.float32)
        return carry
    lax.fori_loop(0, ZROWS, zrow, 0)
    for k in range(ROWS_PS // ZROWS):
        pltpu.sync_copy(rows0_v.at[pl.ds(0, ZROWS)],
                        agg_s.at[pl.ds(s * ROWS_PS + k * ZROWS, ZROWS)])

    pltpu.sync_copy(src_hbm.at[s], src_v)
    pltpu.sync_copy(dst_hbm.at[s], dst_v)
    pltpu.sync_copy(adj_hbm.at[s], adj_v)
    plsc.subcore_barrier()

    def core_body(seq_plane):
        rows = (rows0_v, rows1_v)
        pltpu.async_copy(seq_plane.at[src_v.at[0]], rows0_v, sem).wait()

        def chunk(i, carry):
            for b in range(2):
                @pl.when(i % 2 == b)
                def _():
                    rb, rnb = rows[b], rows[1 - b]
                    dogather = i + 1 < NCHUNK

                    @pl.when(dogather)
                    def _():
                        pltpu.async_copy(
                            seq_plane.at[src_v.at[i + 1]], rnb, sem)

                    def edge(e, carry2):
                        a = plsc.load_gather(
                            adj_v, [jnp.full((16,), i * CH, jnp.int32) + e])
                        for j in range(DH // 16):
                            sl = pl.ds(j * 16, 16)
                            rb[e, sl] = rb[e, sl] * a
                        return carry2
                    lax.fori_loop(0, CH, edge, 0)
                    pltpu.sync_copy(rb, agg_s.at[dst_v.at[i]], add=True)

                    @pl.when(dogather)
                    def _():
                        pltpu.make_async_copy(
                            seq_plane.at[src_v.at[i + 1]], rnb, sem).wait()
            return carry
        lax.fori_loop(0, NCHUNK, chunk, 0)

    @pl.when(c == 0)
    def _():
        core_body(seq_hbm.at[0])

    @pl.when(c == 1)
    def _():
        core_body(seq_hbm.at[1])

    plsc.subcore_barrier()

    @pl.when(c == 0)
    def _():
        pltpu.sync_copy(agg_s.at[pl.ds(s * ROWS_PS, ROWS_PS)],
                        out_hbm.at[0].at[pl.ds(s * ROWS_PS, ROWS_PS)])

    @pl.when(c == 1)
    def _():
        pltpu.sync_copy(agg_s.at[pl.ds(s * ROWS_PS, ROWS_PS)],
                        out_hbm.at[1].at[pl.ds(s * ROWS_PS, ROWS_PS)])


def _sc_aggregate(seq, src, dst, adj):
    mesh = plsc.VectorSubcoreMesh(core_axis_name="c", subcore_axis_name="s")
    return pl.kernel(
        _sc_body,
        out_type=jax.ShapeDtypeStruct((2, N, DH), jnp.float32),
        mesh=mesh,
        scratch_types=[
            pltpu.VMEM((NCHUNK, CH), jnp.int32),
            pltpu.VMEM((NCHUNK, CH), jnp.int32),
            pltpu.VMEM((EPS,), jnp.float32),
            pltpu.VMEM((CH, DH), jnp.float32),
            pltpu.VMEM((CH, DH), jnp.float32),
            pltpu.VMEM_SHARED((N, DH), jnp.float32),
            pltpu.SemaphoreType.DMA,
        ],
        compiler_params=pltpu.CompilerParams(use_tc_tiling_on_sc=False,
                                             needs_layout_passes=False),
    )(seq, src, dst, adj)


def _act_body(agg_ref, b_ref, a_ref, o_ref):
    x = jnp.concatenate([agg_ref[0], agg_ref[1]], axis=1) + b_ref[...]
    alpha = a_ref[0]
    o_ref[...] = jnp.where(x >= 0, x, alpha * x)


def _epilogue(agg, bias, prelu_a):
    return pl.pallas_call(
        _act_body,
        grid=(N // MM_BLK,),
        in_specs=[
            pl.BlockSpec((2, MM_BLK, DH), lambda i: (0, i, 0)),
            pl.BlockSpec((1, D), lambda i: (0, 0)),
            pl.BlockSpec(memory_space=pltpu.SMEM),
        ],
        out_specs=pl.BlockSpec((MM_BLK, D), lambda i: (i, 0)),
        out_shape=jax.ShapeDtypeStruct((N, D), jnp.float32),
    )(agg, bias.reshape(1, D), prelu_a)


@jax.jit
def kernel(features, edge_index, adj_values, W, bias, prelu_a):
    src = edge_index[0].reshape(NS, NCHUNK, CH)
    dst = edge_index[1].reshape(NS, NCHUNK, CH)
    adj = adj_values.reshape(NS, EPS)
    seq = _matmul_halves(features, W)
    agg = _sc_aggregate(seq, src, dst, adj)
    return _epilogue(agg, bias, prelu_a)


# speedup vs baseline: 6.2655x; 1.0454x over previous
import functools

import jax
import jax.numpy as jnp
from jax import lax
from jax.experimental import pallas as pl
from jax.experimental.pallas import tpu as pltpu
from jax.experimental.pallas import tpu_sc as plsc

N = 10000
E = 160000
D = 256
DH = 128
NS = 16
EPS = E // NS
CH = 80
NCHUNK = EPS // CH
ROWS_PS = N // NS
ZROWS = 25
MM_BLK = 1000


def _mm_body(x_ref, w_ref, o_ref):
    r = lax.dot_general(x_ref[...], w_ref[...], (((1,), (1,)), ((), ())),
                        preferred_element_type=jnp.float32)
    o_ref[0] = r[:, :DH]
    o_ref[1] = r[:, DH:]


def _matmul_halves(features, W):
    return pl.pallas_call(
        _mm_body,
        grid=(N // MM_BLK,),
        in_specs=[
            pl.BlockSpec((MM_BLK, D), lambda i: (i, 0)),
            pl.BlockSpec((D, D), lambda i: (0, 0)),
        ],
        out_specs=pl.BlockSpec((2, MM_BLK, DH), lambda i: (0, i, 0)),
        out_shape=jax.ShapeDtypeStruct((2, N, DH), jnp.float32),
    )(features, W)


def _sc_body(seq_hbm, src_hbm, dst_hbm, adj_hbm, out_hbm,
             src_v, dst_v, adj_v, rows0_v, rows1_v, agg_s, gsem, ssem):
    c = lax.axis_index("c")
    s = lax.axis_index("s")

    def zrow(r, carry):
        for j in range(DH // 16):
            rows0_v[r, pl.ds(j * 16, 16)] = jnp.zeros((16,), jnp.float32)
        return carry
    lax.fori_loop(0, ZROWS, zrow, 0)
    for k in range(ROWS_PS // ZROWS):
        pltpu.sync_copy(rows0_v.at[pl.ds(0, ZROWS)],
                        agg_s.at[pl.ds(s * ROWS_PS + k * ZROWS, ZROWS)])

    pltpu.sync_copy(src_hbm.at[s], src_v)
    pltpu.sync_copy(dst_hbm.at[s], dst_v)
    pltpu.sync_copy(adj_hbm.at[s], adj_v)
    plsc.subcore_barrier()

    def core_body(seq_plane):
        rows = (rows0_v, rows1_v)
        pltpu.async_copy(seq_plane.at[src_v.at[0]], rows0_v, gsem).wait()

        def chunk(i, carry):
            for b in range(2):
                @pl.when(i % 2 == b)
                def _():
                    rb, rnb = rows[b], rows[1 - b]
                    dogather = i + 1 < NCHUNK

                    @pl.when(i >= 1)
                    def _():
                        pltpu.make_async_copy(
                            rnb, agg_s.at[dst_v.at[i - 1]], ssem).wait()

                    @pl.when(dogather)
                    def _():
                        pltpu.async_copy(
                            seq_plane.at[src_v.at[i + 1]], rnb, gsem)

                    def edge(e, carry2):
                        a = plsc.load_gather(
                            adj_v, [jnp.full((16,), i * CH, jnp.int32) + e])
                        for j in range(DH // 16):
                            sl = pl.ds(j * 16, 16)
                            rb[e, sl] = rb[e, sl] * a
                        return carry2
                    lax.fori_loop(0, CH, edge, 0, unroll=4)
                    pltpu.async_copy(rb, agg_s.at[dst_v.at[i]], ssem,
                                     add=True)

                    @pl.when(dogather)
                    def _():
                        pltpu.make_async_copy(
                            seq_plane.at[src_v.at[i + 1]], rnb, gsem).wait()
            return carry
        lax.fori_loop(0, NCHUNK, chunk, 0)
        pltpu.make_async_copy(rows[(NCHUNK - 1) % 2],
                              agg_s.at[dst_v.at[NCHUNK - 1]], ssem).wait()

    @pl.when(c == 0)
    def _():
        core_body(seq_hbm.at[0])

    @pl.when(c == 1)
    def _():
        core_body(seq_hbm.at[1])

    plsc.subcore_barrier()

    @pl.when(c == 0)
    def _():
        pltpu.sync_copy(agg_s.at[pl.ds(s * ROWS_PS, ROWS_PS)],
                        out_hbm.at[0].at[pl.ds(s * ROWS_PS, ROWS_PS)])

    @pl.when(c == 1)
    def _():
        pltpu.sync_copy(agg_s.at[pl.ds(s * ROWS_PS, ROWS_PS)],
                        out_hbm.at[1].at[pl.ds(s * ROWS_PS, ROWS_PS)])


def _sc_aggregate(seq, src, dst, adj):
    mesh = plsc.VectorSubcoreMesh(core_axis_name="c", subcore_axis_name="s")
    return pl.kernel(
        _sc_body,
        out_type=jax.ShapeDtypeStruct((2, N, DH), jnp.float32),
        mesh=mesh,
        scratch_types=[
            pltpu.VMEM((NCHUNK, CH), jnp.int32),
            pltpu.VMEM((NCHUNK, CH), jnp.int32),
            pltpu.VMEM((EPS,), jnp.float32),
            pltpu.VMEM((CH, DH), jnp.float32),
            pltpu.VMEM((CH, DH), jnp.float32),
            pltpu.VMEM_SHARED((N, DH), jnp.float32),
            pltpu.SemaphoreType.DMA,
            pltpu.SemaphoreType.DMA,
        ],
        compiler_params=pltpu.CompilerParams(use_tc_tiling_on_sc=False,
                                             needs_layout_passes=False),
    )(seq, src, dst, adj)


def _act_body(agg_ref, b_ref, a_ref, o_ref):
    x = jnp.concatenate([agg_ref[0], agg_ref[1]], axis=1) + b_ref[...]
    alpha = a_ref[0]
    o_ref[...] = jnp.where(x >= 0, x, alpha * x)


def _epilogue(agg, bias, prelu_a):
    return pl.pallas_call(
        _act_body,
        grid=(N // MM_BLK,),
        in_specs=[
            pl.BlockSpec((2, MM_BLK, DH), lambda i: (0, i, 0)),
            pl.BlockSpec((1, D), lambda i: (0, 0)),
            pl.BlockSpec(memory_space=pltpu.SMEM),
        ],
        out_specs=pl.BlockSpec((MM_BLK, D), lambda i: (i, 0)),
        out_shape=jax.ShapeDtypeStruct((N, D), jnp.float32),
    )(agg, bias.reshape(1, D), prelu_a)


@jax.jit
def kernel(features, edge_index, adj_values, W, bias, prelu_a):
    src = edge_index[0].reshape(NS, NCHUNK, CH)
    dst = edge_index[1].reshape(NS, NCHUNK, CH)
    adj = adj_values.reshape(NS, EPS)
    seq = _matmul_halves(features, W)
    agg = _sc_aggregate(seq, src, dst, adj)
    return _epilogue(agg, bias, prelu_a)
